# baseline (speedup 1.0000x reference)
import functools

import jax
import jax.numpy as jnp
from jax import lax
from jax.experimental import pallas as pl
from jax.experimental.pallas import tpu as pltpu
from jax.experimental.pallas import tpu_sc as plsc

B, S, D = 1024, 200, 64
VOCAB = 1000000

_info = plsc.get_sparse_core_info()
NC, NS = _info.num_cores, _info.num_subcores
NW = NC * NS
SEQ_W = B // NW
ROWS_W = SEQ_W * S

_G = (S + 15) // 16

_mesh = plsc.VectorSubcoreMesh(core_axis_name="c", subcore_axis_name="s")


@functools.partial(
    pl.kernel,
    out_type=jax.ShapeDtypeStruct((B * S // 2, 2 * D), jnp.float32),
    mesh=_mesh,
    compiler_params=pltpu.CompilerParams(needs_layout_passes=False),
    scratch_types=[
        pltpu.VMEM((ROWS_W + 16,), jnp.int32),
        pltpu.VMEM((S // 2, 2 * D), jnp.float32),
        pltpu.VMEM((S, 2 * D), jnp.float32),
        pltpu.VMEM((S, 2 * D), jnp.float32),
        pltpu.VMEM((S, 2 * D), jnp.float32),
        pltpu.VMEM((S, 2 * D), jnp.float32),
        pltpu.VMEM((_G * 16,), jnp.int32),
        pltpu.VMEM((_G * 16,), jnp.int32),
        pltpu.VMEM((_G * 16,), jnp.int32),
        pltpu.VMEM((_G * 16,), jnp.int32),
        pltpu.SemaphoreType.DMA,
        pltpu.SemaphoreType.DMA,
        pltpu.SemaphoreType.DMA,
        pltpu.SemaphoreType.DMA,
    ],
)
def _embed(x_hbm, tok_hbm, pos_hbm, out_hbm, idx_v, pos_v, buf0, buf1,
           ob0, ob1, jdx0, jdx1, par0, par1, gsem0, gsem1, ssem0, ssem1):
    wid = lax.axis_index("s") * NC + lax.axis_index("c")
    base = pl.multiple_of(wid * ROWS_W, ROWS_W)

    pltpu.sync_copy(pos_hbm, pos_v)
    pltpu.sync_copy(x_hbm.at[pl.ds(base, ROWS_W)], idx_v.at[pl.ds(0, ROWS_W)])

    bufs = (buf0, buf1)
    obs = (ob0, ob1)
    jdxs = (jdx0, jdx1)
    pars = (par0, par1)
    gsems = (gsem0, gsem1)
    ssems = (ssem0, ssem1)

    def prep_indices(s):
        b = s % 2
        jdx, par = jdxs[b], pars[b]

        def body(g, carry):
            sl = pl.ds(s * S + g * 16, 16)
            t = idx_v[sl]
            jdx[pl.ds(g * 16, 16)] = lax.shift_right_logical(t, 1)
            par[pl.ds(g * 16, 16)] = lax.bitwise_and(t, 1)
            return carry
        lax.fori_loop(0, _G, body, 0)

    def start_gather(s):
        b = s % 2
        d0 = pltpu.async_copy(
            tok_hbm.at[jdxs[b].at[pl.ds(0, 128)]],
            bufs[b].at[pl.ds(0, 128)], gsems[b])
        d1 = pltpu.async_copy(
            tok_hbm.at[jdxs[b].at[pl.ds(128, S - 128)]],
            bufs[b].at[pl.ds(128, S - 128)], gsems[b])
        return (d0, d1)

    def compute(s):
        b = s % 2
        buf, ob, par = bufs[b], obs[(s // 2) % 2], pars[b]
        q = (s % 2) * (S // 2)

        def gbody(g, carry):
            r0 = g * 16
            rvec = r0 + lax.iota(jnp.int32, 16)
            mask = rvec < S
            pvec = par[pl.ds(r0, 16)] * D
            rv2 = lax.shift_right_logical(rvec, 1)
            cb2 = lax.bitwise_and(rvec, 1) * D

            def dbody(d, carry2):
                dv = jnp.full((16,), 0, jnp.int32) + d
                c2 = cb2 + dv
                tok = plsc.load_gather(buf, [rvec, pvec + dv], mask=mask)
                pp = plsc.load_gather(pos_v, [rv2, c2], mask=mask)
                plsc.store_scatter(ob, [q + rv2, c2], tok + pp, mask=mask)
                return carry2
            lax.fori_loop(0, D, dbody, 0)
            return carry
        lax.fori_loop(0, _G, gbody, 0)

    gd = [None, None]
    sd = [None, None]
    for s in range(SEQ_W + 1):
        if s < SEQ_W:
            b = s % 2
            prep_indices(s)
            gd[b] = start_gather(s)
        if s >= 1:
            sp = s - 1
            bp = sp % 2
            if sp % 2 == 0 and sd[(sp // 2) % 2] is not None:
                sd[(sp // 2) % 2].wait()
            for d in gd[bp]:
                d.wait()
            compute(sp)
            if sp % 2 == 1:
                k = sp // 2
                sd[k % 2] = pltpu.async_copy(
                    obs[k % 2],
                    out_hbm.at[pl.ds(pl.multiple_of(base // 2 + k * S, 8), S)],
                    ssems[k % 2])
    sd[0].wait()
    sd[1].wait()


def kernel(x, token_table, pos_table):
    xf = x.reshape(B * S)
    tt2 = token_table.reshape(VOCAB // 2, 2 * D)
    pos2 = pos_table.reshape(S // 2, 2 * D)
    out = _embed(xf, tt2, pos2)
    return out.reshape(B, S, D)


# speedup vs baseline: 1.5955x; 1.5955x over previous
import functools

import jax
import jax.numpy as jnp
from jax import lax
from jax.experimental import pallas as pl
from jax.experimental.pallas import tpu as pltpu
from jax.experimental.pallas import tpu_sc as plsc

B, S, D = 1024, 200, 64

_info = plsc.get_sparse_core_info()
NC, NS = _info.num_cores, _info.num_subcores
NW = NC * NS
BW = B // NW
SC_ = 8
NCHUNK = S // SC_
CT = SC_ * BW

_mesh = plsc.VectorSubcoreMesh(core_axis_name="c", subcore_axis_name="s")


@functools.partial(
    pl.kernel,
    out_type=jax.ShapeDtypeStruct((S, 8, 8, 8, 128), jnp.float32),
    mesh=_mesh,
    compiler_params=pltpu.CompilerParams(
        use_tc_tiling_on_sc=False, needs_layout_passes=False),
    scratch_types=[
        pltpu.VMEM((BW * S,), jnp.int32),
        pltpu.VMEM((S, D), jnp.float32),
        pltpu.VMEM((CT, D), jnp.float32),
        pltpu.VMEM((CT, D), jnp.float32),
        pltpu.VMEM((SC_, 8, 8, BW), jnp.float32),
        pltpu.VMEM((SC_, 8, 8, BW), jnp.float32),
        pltpu.VMEM((CT,), jnp.int32),
        pltpu.VMEM((CT,), jnp.int32),
        pltpu.SemaphoreType.DMA,
        pltpu.SemaphoreType.DMA,
        pltpu.SemaphoreType.DMA,
        pltpu.SemaphoreType.DMA,
    ],
)
def _embed(x_hbm, tok_hbm, pos_hbm, out_hbm, idx_v, pos_v, gb0, gb1,
           vb0, vb1, cl0, cl1, gsem0, gsem1, ssem0, ssem1):
    wid = lax.axis_index("s") * NC + lax.axis_index("c")
    bt0 = wid // 4
    bi0 = (wid % 4) * BW

    pltpu.sync_copy(pos_hbm, pos_v)
    pltpu.sync_copy(x_hbm.at[pl.ds(wid * BW * S, BW * S)], idx_v)

    gbs = (gb0, gb1)
    vbs = (vb0, vb1)
    cls = (cl0, cl1)
    gsems = (gsem0, gsem1)
    ssems = (ssem0, ssem1)

    lanes = lax.iota(jnp.int32, 16)
    apat_idx = lanes * S
    dt_vecs = [lax.shift_right_logical(lanes, 3) + 2 * k for k in range(4)]
    di_vec = lax.bitwise_and(lanes, 7)

    def prep_clist(c):
        cl = cls[c % 2]
        s0 = c * SC_

        def body(g, carry):
            addr = apat_idx + ((g % 2) * 16 * S + s0 + g // 2)
            cl[pl.ds(g * 16, 16)] = plsc.load_gather(idx_v, [addr])
            return carry
        lax.fori_loop(0, CT // 16, body, 0)

    def start_gather(c):
        b = c % 2
        d0 = pltpu.async_copy(
            tok_hbm.at[cls[b].at[pl.ds(0, 128)]],
            gbs[b].at[pl.ds(0, 128)], gsems[b])
        d1 = pltpu.async_copy(
            tok_hbm.at[cls[b].at[pl.ds(128, 128)]],
            gbs[b].at[pl.ds(128, 128)], gsems[b])
        return (d0, d1)

    def compute(c):
        b = c % 2
        gb, vb = gbs[b], vbs[b]
        s0 = c * SC_

        def sbody(sl, carry):
            pos_regs = [pos_v[s0 + sl, pl.ds(k * 16, 16)] for k in range(4)]
            slv = jnp.full((16,), 0, jnp.int32) + sl

            def bbody(bl, carry2):
                i = sl * BW + bl
                blv = jnp.full((16,), 0, jnp.int32) + bl
                for k in range(4):
                    v = gb[i, pl.ds(k * 16, 16)] + pos_regs[k]
                    plsc.store_scatter(vb, [slv, dt_vecs[k], di_vec, blv], v)
                return carry2
            lax.fori_loop(0, BW, bbody, 0)
            return carry
        lax.fori_loop(0, SC_, sbody, 0)

    gd = [None, None]
    sd = [None, None]
    for c in range(NCHUNK + 1):
        if c < NCHUNK:
            b = c % 2
            if sd[b] is not None:
                sd[b].wait()
            prep_clist(c)
            gd[b] = start_gather(c)
        if c >= 1:
            cp = c - 1
            bp = cp % 2
            for d in gd[bp]:
                d.wait()
            compute(cp)
            sd[bp] = pltpu.async_copy(
                vbs[bp],
                out_hbm.at[pl.ds(cp * SC_, SC_), :, bt0, :, pl.ds(bi0, BW)],
                ssems[bp])
    sd[(NCHUNK - 2) % 2].wait()
    sd[(NCHUNK - 1) % 2].wait()


def kernel(x, token_table, pos_table):
    xf = x.reshape(B * S).astype(jnp.int32)
    out5 = _embed(xf, token_table, pos_table)
    out = out5.transpose(0, 1, 3, 2, 4).reshape(S, D, B).transpose(2, 0, 1)
    return out


# speedup vs baseline: 1.8070x; 1.1326x over previous
import functools

import jax
import jax.numpy as jnp
from jax import lax
from jax.experimental import pallas as pl
from jax.experimental.pallas import tpu as pltpu
from jax.experimental.pallas import tpu_sc as plsc

B, S, D = 1024, 200, 64

_info = plsc.get_sparse_core_info()
NC, NS = _info.num_cores, _info.num_subcores
NW = NC * NS
BW = B // NW
SC_ = 8
NCHUNK = S // SC_
CT = SC_ * BW

_mesh = plsc.VectorSubcoreMesh(core_axis_name="c", subcore_axis_name="s")


@functools.partial(
    pl.kernel,
    out_type=jax.ShapeDtypeStruct((S, 8, 8, 8, 128), jnp.float32),
    mesh=_mesh,
    compiler_params=pltpu.CompilerParams(
        use_tc_tiling_on_sc=False, needs_layout_passes=False),
    scratch_types=[
        pltpu.VMEM((BW * S,), jnp.int32),
        pltpu.VMEM((S, D), jnp.float32),
        pltpu.VMEM((CT, D), jnp.float32),
        pltpu.VMEM((CT, D), jnp.float32),
        pltpu.VMEM((SC_, 8, 8, BW), jnp.float32),
        pltpu.VMEM((SC_, 8, 8, BW), jnp.float32),
        pltpu.VMEM((CT,), jnp.int32),
        pltpu.VMEM((CT,), jnp.int32),
        pltpu.SemaphoreType.DMA,
        pltpu.SemaphoreType.DMA,
        pltpu.SemaphoreType.DMA,
        pltpu.SemaphoreType.DMA,
    ],
)
def _embed(x_hbm, tok_hbm, pos_hbm, out_hbm, idx_v, pos_v, gb0, gb1,
           vb0, vb1, cl0, cl1, gsem0, gsem1, ssem0, ssem1):
    wid = lax.axis_index("s") * NC + lax.axis_index("c")
    bt0 = wid // 4
    bi0 = (wid % 4) * BW

    pltpu.sync_copy(pos_hbm, pos_v)
    pltpu.sync_copy(x_hbm.at[pl.ds(wid * BW * S, BW * S)], idx_v)

    gbs = (gb0, gb1)
    vbs = (vb0, vb1)
    cls = (cl0, cl1)
    gsems = (gsem0, gsem1)
    ssems = (ssem0, ssem1)

    lanes = lax.iota(jnp.int32, 16)
    apat_idx = lanes * S
    dt_vecs = [lax.shift_right_logical(lanes, 3) + 2 * k for k in range(4)]
    di_vec = lax.bitwise_and(lanes, 7)
    zero_vec = lanes * 0

    def prep_clist(c):
        cl = cls[c % 2]
        s0 = c * SC_

        def body(g, carry):
            addr = apat_idx + ((g % 2) * 16 * S + s0 + g // 2)
            cl[pl.ds(g * 16, 16)] = plsc.load_gather(idx_v, [addr])
            return carry
        lax.fori_loop(0, CT // 16, body, 0)

    def start_gather(c):
        b = c % 2
        d0 = pltpu.async_copy(
            tok_hbm.at[cls[b].at[pl.ds(0, 128)]],
            gbs[b].at[pl.ds(0, 128)], gsems[b])
        d1 = pltpu.async_copy(
            tok_hbm.at[cls[b].at[pl.ds(128, 128)]],
            gbs[b].at[pl.ds(128, 128)], gsems[b])
        return (d0, d1)

    def compute(c):
        b = c % 2
        gb, vb = gbs[b], vbs[b]
        s0 = c * SC_

        def sbody(sl, carry):
            pos_regs = [pos_v[s0 + sl, pl.ds(k * 16, 16)] for k in range(4)]
            slv = jnp.full((16,), 0, jnp.int32) + sl

            def bbody(bl):
                i = sl * BW + bl
                blv = jnp.full((16,), 0, jnp.int32) + bl
                for k in range(4):
                    v = gb[i, pl.ds(k * 16, 16)] + pos_regs[k]
                    plsc.store_scatter(vb, [slv, dt_vecs[k], di_vec, blv], v)
            plsc.parallel_loop(0, BW, 1, unroll=4)(bbody)
            return carry
        lax.fori_loop(0, SC_, sbody, 0)

    gd = [None, None]
    sd = [None, None]
    for c in range(NCHUNK + 1):
        if c < NCHUNK:
            b = c % 2
            if sd[b] is not None:
                sd[b].wait()
            prep_clist(c)
            gd[b] = start_gather(c)
        if c >= 1:
            cp = c - 1
            bp = cp % 2
            for d in gd[bp]:
                d.wait()
            compute(cp)
            sd[bp] = pltpu.async_copy(
                vbs[bp],
                out_hbm.at[pl.ds(cp * SC_, SC_), :, bt0, :, pl.ds(bi0, BW)],
                ssems[bp])
    sd[(NCHUNK - 2) % 2].wait()
    sd[(NCHUNK - 1) % 2].wait()


def kernel(x, token_table, pos_table):
    xf = x.reshape(B * S).astype(jnp.int32)
    out5 = _embed(xf, token_table, pos_table)
    out = out5.transpose(0, 1, 3, 2, 4).reshape(S, D, B).transpose(2, 0, 1)
    return out
